# baseline (speedup 1.0000x reference)
import functools

import jax
import jax.numpy as jnp
from jax import lax
from jax.experimental import pallas as pl
from jax.experimental.pallas import tpu as pltpu
from jax.experimental.pallas import tpu_sc as plsc

B, S, D = 4, 4096, 1024
NC, NS, L = 2, 16, 16
NW = NC * NS
SPW = S // NW
T = 16
NCH = SPW // T
CW = T * D

_mesh = plsc.VectorSubcoreMesh(
    core_axis_name="c", subcore_axis_name="s", num_cores=NC, num_subcores=NS
)


@functools.partial(
    pl.kernel,
    out_type=jax.ShapeDtypeStruct((B, S * D), jnp.float32),
    mesh=_mesh,
    scratch_types=[
        pltpu.VMEM((CW,), jnp.float32),
        pltpu.VMEM((CW,), jnp.float32),
    ],
)
def _sc_add(x_hbm, pos_hbm, out_hbm, p_v, x_v):
    wid = lax.axis_index("s") * NC + lax.axis_index("c")
    e0 = wid * (SPW * D)

    def chunk_body(ci, carry):
        e = e0 + ci * CW
        pltpu.sync_copy(pos_hbm.at[pl.ds(e, CW)], p_v)

        def batch_body(b, carry2):
            pltpu.sync_copy(x_hbm.at[b, pl.ds(e, CW)], x_v)

            @plsc.parallel_loop(0, CW // L, unroll=8)
            def _acc(i):
                sl = pl.ds(i * L, L)
                plsc.addupdate(x_v.at[sl], p_v[sl])

            pltpu.sync_copy(x_v, out_hbm.at[b, pl.ds(e, CW)])
            return carry2

        lax.fori_loop(0, B, batch_body, 0)
        return carry

    lax.fori_loop(0, NCH, chunk_body, 0)


def kernel(x, pos_table):
    out = _sc_add(x.reshape(B, S * D), pos_table.reshape(S * D))
    return out.reshape(B, S, D)


# speedup vs baseline: 1.2679x; 1.2679x over previous
import functools

import jax
import jax.numpy as jnp
from jax import lax
from jax.experimental import pallas as pl
from jax.experimental.pallas import tpu as pltpu
from jax.experimental.pallas import tpu_sc as plsc

B, S, D = 4, 4096, 1024
NC, NS, L = 2, 16, 16
NW = NC * NS
SPW = S // NW
T = 16
NCH = SPW // T
CW = T * D
NSTEP = NCH * B

_mesh = plsc.VectorSubcoreMesh(
    core_axis_name="c", subcore_axis_name="s", num_cores=NC, num_subcores=NS
)


@functools.partial(
    pl.kernel,
    out_type=jax.ShapeDtypeStruct((B, S * D), jnp.float32),
    mesh=_mesh,
    scratch_types=[
        [pltpu.VMEM((CW,), jnp.float32)] * 2,
        [pltpu.VMEM((CW,), jnp.float32)] * 4,
        [pltpu.SemaphoreType.DMA] * 2,
        [pltpu.SemaphoreType.DMA] * 4,
        [pltpu.SemaphoreType.DMA] * 4,
    ],
)
def _sc_add(x_hbm, pos_hbm, out_hbm, p_v, x_v, sem_p, sem_x, sem_o):
    wid = lax.axis_index("s") * NC + lax.axis_index("c")
    e0 = wid * (SPW * D)

    def pos_load(ci):
        return pltpu.make_async_copy(
            pos_hbm.at[pl.ds(e0 + ci * CW, CW)], p_v[ci % 2], sem_p[ci % 2]
        )

    def x_load(step):
        ci, b = step // B, step % B
        return pltpu.make_async_copy(
            x_hbm.at[b, pl.ds(e0 + ci * CW, CW)], x_v[step % 4], sem_x[step % 4]
        )

    def out_store(step):
        ci, b = step // B, step % B
        return pltpu.make_async_copy(
            x_v[step % 4], out_hbm.at[b, pl.ds(e0 + ci * CW, CW)], sem_o[step % 4]
        )

    pos_load(0).start()
    for s in range(3):
        x_load(s).start()

    for step in range(NSTEP):
        ci, b = step // B, step % B
        if b == 0:
            pos_load(ci).wait()
            if ci + 1 < NCH:
                pos_load(ci + 1).start()
        if step + 3 < NSTEP:
            if step >= 1:
                out_store(step - 1).wait()
            x_load(step + 3).start()
        x_load(step).wait()

        pv = p_v[ci % 2]
        xv = x_v[step % 4]

        @plsc.parallel_loop(0, CW // L, unroll=8)
        def _acc(i):
            sl = pl.ds(i * L, L)
            plsc.addupdate(xv.at[sl], pv[sl])

        out_store(step).start()

    for step in range(NSTEP - 4, NSTEP):
        out_store(step).wait()


def kernel(x, pos_table):
    out = _sc_add(x.reshape(B, S * D), pos_table.reshape(S * D))
    return out.reshape(B, S, D)


# speedup vs baseline: 3.0731x; 2.4238x over previous
import functools

import jax
import jax.numpy as jnp
from jax import lax
from jax.experimental import pallas as pl
from jax.experimental.pallas import tpu as pltpu
from jax.experimental.pallas import tpu_sc as plsc

B, S, D = 4, 4096, 1024
NC, NS, L = 2, 16, 16
NW = NC * NS
SPW = S // NW
T = 16
NCH = SPW // T
NSTEP = NCH * B
NVEC = T * D // L

_mesh = plsc.VectorSubcoreMesh(
    core_axis_name="c", subcore_axis_name="s", num_cores=NC, num_subcores=NS
)


@functools.partial(
    pl.kernel,
    out_type=jax.ShapeDtypeStruct((B, S, D), jnp.float32),
    mesh=_mesh,
    compiler_params=pltpu.CompilerParams(use_tc_tiling_on_sc=True),
    scratch_types=[
        [pltpu.VMEM((T, D), jnp.float32)] * 2,
        [pltpu.VMEM((T, D), jnp.float32)] * 4,
        [pltpu.SemaphoreType.DMA] * 2,
        [pltpu.SemaphoreType.DMA] * 4,
        [pltpu.SemaphoreType.DMA] * 4,
    ],
)
def _sc_add(x_hbm, pos_hbm, out_hbm, p_v, x_v, sem_p, sem_x, sem_o):
    wid = lax.axis_index("s") * NC + lax.axis_index("c")
    s0 = wid * SPW

    def pos_load(ci):
        return pltpu.make_async_copy(
            pos_hbm.at[pl.ds(s0 + ci * T, T)], p_v[ci % 2], sem_p[ci % 2]
        )

    def x_load(step):
        ci, b = step // B, step % B
        return pltpu.make_async_copy(
            x_hbm.at[b, pl.ds(s0 + ci * T, T)], x_v[step % 4], sem_x[step % 4]
        )

    def out_store(step):
        ci, b = step // B, step % B
        return pltpu.make_async_copy(
            x_v[step % 4], out_hbm.at[b, pl.ds(s0 + ci * T, T)], sem_o[step % 4]
        )

    pos_load(0).start()
    for s in range(3):
        x_load(s).start()

    for step in range(NSTEP):
        ci, b = step // B, step % B
        if b == 0:
            pos_load(ci).wait()
            if ci + 1 < NCH:
                pos_load(ci + 1).start()
        if step + 3 < NSTEP:
            if step >= 1:
                out_store(step - 1).wait()
            x_load(step + 3).start()
        x_load(step).wait()

        pv = p_v[ci % 2]
        xv = x_v[step % 4]

        @plsc.parallel_loop(0, NVEC, unroll=8)
        def _acc(i):
            r = i // (D // L)
            c = (i % (D // L)) * L
            sl = pl.ds(c, L)
            plsc.addupdate(xv.at[r, sl], pv[r, sl])

        out_store(step).start()

    for step in range(NSTEP - 4, NSTEP):
        out_store(step).wait()


def kernel(x, pos_table):
    return _sc_add(x, pos_table)
